# baseline (speedup 1.0000x reference)
import functools

import jax
import jax.numpy as jnp
from jax import lax
from jax.experimental import pallas as pl
from jax.experimental.pallas import tpu as pltpu
from jax.experimental.pallas import tpu_sc as plsc

B = 4096
F = 26
V = 100000
D = 16

NC = 2
NS = 16
NW = NC * NS
BPW = B // NW
NCHUNK = BPW // 16


def _sc_body(idx_hbm, e2_hbm, w2_hbm, lin_hbm, parts_hbm,
             idx_v, rows_v, wv_v, out_v, part_v, sem_e, sem_w):
    c = lax.axis_index("c")
    s = lax.axis_index("s")
    wid = s * NC + c
    base = wid * BPW

    pltpu.sync_copy(idx_hbm.at[wid], idx_v)
    cps_e, cps_w = [], []
    for f in range(F):
        cps_e.append(pltpu.async_copy(e2_hbm.at[idx_v.at[f]], rows_v.at[f], sem_e))
        cps_w.append(pltpu.async_copy(w2_hbm.at[idx_v.at[f]], wv_v.at[f], sem_w))
    for cp in cps_e:
        cp.wait()

    def body(bb, carry):
        p_acc, q_acc = carry
        e0 = rows_v[0, bb, :]
        s_v = e0
        q_v = e0 * e0
        for f in range(1, F):
            e = rows_v[f, bb, :]
            s_v = s_v + e
            q_v = q_v + e * e
        return (p_acc + s_v * s_v, q_acc + q_v)

    zero = jnp.zeros((16,), jnp.float32)
    p_acc, q_acc = lax.fori_loop(0, BPW, body, (zero, zero))
    part_v[...] = p_acc - q_acc
    pltpu.sync_copy(part_v, parts_hbm.at[wid])

    for cp in cps_w:
        cp.wait()
    for ci in range(NCHUNK):
        acc = wv_v[0, pl.ds(ci * 16, 16)]
        for f in range(1, F):
            acc = acc + wv_v[f, pl.ds(ci * 16, 16)]
        out_v[pl.ds(ci * 16, 16)] = acc
    pltpu.sync_copy(out_v, lin_hbm.at[pl.ds(base, BPW)])


@functools.partial(
    pl.kernel,
    out_type=(
        jax.ShapeDtypeStruct((B,), jnp.float32),
        jax.ShapeDtypeStruct((NW, 16), jnp.float32),
    ),
    mesh=plsc.VectorSubcoreMesh(core_axis_name="c", subcore_axis_name="s"),
    compiler_params=pltpu.CompilerParams(use_tc_tiling_on_sc=False),
    scratch_types=[
        pltpu.VMEM((F, BPW), jnp.int32),
        pltpu.VMEM((F, BPW, D), jnp.float32),
        pltpu.VMEM((F, BPW), jnp.float32),
        pltpu.VMEM((BPW,), jnp.float32),
        pltpu.VMEM((16,), jnp.float32),
        pltpu.SemaphoreType.DMA,
        pltpu.SemaphoreType.DMA,
    ],
)
def _sc_gather_fm(idx_hbm, e2_hbm, w2_hbm, lin_hbm, parts_hbm,
                  idx_v, rows_v, wv_v, out_v, part_v, sem_e, sem_w):
    _sc_body(idx_hbm, e2_hbm, w2_hbm, lin_hbm, parts_hbm,
             idx_v, rows_v, wv_v, out_v, part_v, sem_e, sem_w)


def _tc_combine(lin_ref, parts_ref, b_ref, out_ref):
    inter = 0.5 * jnp.sum(parts_ref[...]) + b_ref[0]
    out_ref[...] = lin_ref[...] + inter


def kernel(inputs, W_lin, b, E):
    idx = inputs.astype(jnp.int32).T + (jnp.arange(F, dtype=jnp.int32) * V)[:, None]
    idx3 = idx.reshape(F, NW, BPW).transpose(1, 0, 2)
    e2 = E.reshape(F * V, D)
    w2 = W_lin.reshape(F * V)

    lin, parts = _sc_gather_fm(idx3, e2, w2)

    out = pl.pallas_call(
        _tc_combine,
        out_shape=jax.ShapeDtypeStruct((B,), jnp.float32),
    )(lin, parts, b)
    return out[:, None]
